# speedup vs baseline: 1.2270x; 1.2270x over previous
import jax
import jax.numpy as jnp
from jax import lax
from jax.experimental import pallas as pl
from jax.experimental.pallas import tpu as pltpu
from jax.experimental.pallas import tpu_sc as plsc

_B, _S, _D = 4, 4096, 1024
_NW = 32
_SPW = _S // _NW
_K = 16
_NCH = _SPW // _K


def _emb_body(tok_ref, emb_ref, pos_ref, out_ref, idx_v, pos_v, emb_v, sem):
    cid = lax.axis_index("core")
    sid = lax.axis_index("subcore")
    wid = sid * 2 + cid
    s_base = wid * _SPW

    pltpu.sync_copy(tok_ref.at[wid], idx_v)

    def chunk(c, carry):
        s0 = s_base + c * _K
        pltpu.sync_copy(pos_ref.at[pl.ds(s0, _K)], pos_v)
        for b in range(_B):
            pltpu.async_copy(emb_ref.at[idx_v.at[b, c]], emb_v, sem).wait()

            def row(r, carry2):
                for j in range(_D // 16):
                    sl = pl.ds(j * 16, 16)
                    emb_v[r, sl] = emb_v[r, sl] + pos_v[r, sl]
                return carry2

            lax.fori_loop(0, _K, row, 0)
            pltpu.sync_copy(emb_v, out_ref.at[b, pl.ds(s0, _K)])
        return carry

    lax.fori_loop(0, _NCH, chunk, 0)


def kernel(tokens, emb_table, pos_table):
    tok = (tokens.astype(jnp.int32)
           .reshape(_B, _NW, _NCH, _K)
           .transpose(1, 0, 2, 3))
    mesh = plsc.VectorSubcoreMesh(core_axis_name="core",
                                  subcore_axis_name="subcore")
    f = pl.kernel(
        _emb_body,
        out_type=jax.ShapeDtypeStruct((_B, _S, _D), jnp.float32),
        mesh=mesh,
        scratch_types=[
            pltpu.VMEM((_B, _NCH, _K), jnp.int32),
            pltpu.VMEM((_K, _D), jnp.float32),
            pltpu.VMEM((_K, _D), jnp.float32),
            pltpu.SemaphoreType.DMA,
        ],
    )
    return f(tok, emb_table, pos_table)
